# baseline (speedup 1.0000x reference)
import functools

import jax
import jax.numpy as jnp
from jax import lax
from jax.experimental import pallas as pl
from jax.experimental.pallas import tpu as pltpu
from jax.experimental.pallas import tpu_sc as plsc

_NW = 32
_CH = 64


def _sc_body(pos_hbm, hid_hbm, emb_hbm, out_hbm, pos_v, buf, ctxbuf, sem):
    B, S, D = hid_hbm.shape
    NCT = emb_hbm.shape[1]
    nseg = _NW // B
    per_seg = S // nseg

    c = lax.axis_index("c")
    s = lax.axis_index("s")
    wid = s * 2 + c
    b = wid // nseg
    seg = wid % nseg

    for j in range(per_seg // _CH):
        r = seg * per_seg + j * _CH
        fetch = pltpu.make_async_copy(hid_hbm.at[b, pl.ds(r, _CH)], buf, sem)
        fetch.start()
        fetch.wait()
        put = pltpu.make_async_copy(buf, out_hbm.at[b, pl.ds(r + NCT, _CH)], sem)
        put.start()
        put.wait()

    @pl.when(seg == 0)
    def _():
        cp = pltpu.make_async_copy(pos_hbm, pos_v, sem)
        cp.start()
        cp.wait()
        cp = pltpu.make_async_copy(emb_hbm.at[pos_v], ctxbuf, sem)
        cp.start()
        cp.wait()
        cp = pltpu.make_async_copy(
            ctxbuf.at[0], out_hbm.at[b, pl.ds(0, NCT)], sem
        )
        cp.start()
        cp.wait()


def _mask_body(mask_ref, mask_out_ref):
    nct = mask_out_ref.shape[2] - mask_ref.shape[2]
    mask_out_ref[0, 0, :nct] = jnp.ones((nct,), mask_out_ref.dtype)
    mask_out_ref[0, 0, nct:] = mask_ref[0, 0]


def kernel(hidden_states, attention_mask, embeddings, position):
    B, S, D = hidden_states.shape
    _, NCT, _ = embeddings.shape
    pos = jnp.asarray(position, jnp.int32).reshape((1,))

    mesh = plsc.VectorSubcoreMesh(core_axis_name="c", subcore_axis_name="s")
    sc_kernel = functools.partial(
        pl.kernel,
        mesh=mesh,
        out_type=jax.ShapeDtypeStruct((B, NCT + S, D), hidden_states.dtype),
        scratch_types=[
            pltpu.VMEM((1,), jnp.int32),
            pltpu.VMEM((_CH, D), hidden_states.dtype),
            pltpu.VMEM((1, NCT, D), hidden_states.dtype),
            pltpu.SemaphoreType.DMA,
        ],
        compiler_params=pltpu.CompilerParams(use_tc_tiling_on_sc=False),
    )(_sc_body)
    out_hid = sc_kernel(pos, hidden_states, embeddings)

    mask3 = attention_mask.reshape(B, 1, S)
    out_mask = pl.pallas_call(
        _mask_body,
        grid=(B,),
        in_specs=[pl.BlockSpec((1, 1, S), lambda b: (b, 0, 0))],
        out_specs=pl.BlockSpec((1, 1, NCT + S), lambda b: (b, 0, 0)),
        out_shape=jax.ShapeDtypeStruct((B, 1, NCT + S), attention_mask.dtype),
    )(mask3)
    return (out_hid, out_mask.reshape(B, NCT + S))
